# baseline (speedup 1.0000x reference)
import jax
import jax.numpy as jnp
from jax.experimental import pallas as pl
from jax.experimental.pallas import tpu as pltpu

_EPS = 1e-12

_NBUF = 4
_ROWS = 128


def _guidance_kernel(p_ref, x_ref, o_ref, buf, sem):
    c = p_ref.shape[1]
    nrows_total = x_ref.shape[0]
    nchunk = nrows_total // _ROWS
    cpb = c // _ROWS

    def copy(k):
        return pltpu.make_async_copy(
            x_ref.at[pl.ds(k * _ROWS, _ROWS), :],
            buf.at[k % _NBUF],
            sem.at[k % _NBUF],
        )

    for k in range(min(_NBUF - 1, nchunk)):
        copy(k).start()

    s = None
    norm2 = None
    for k in range(nchunk):
        copy(k).wait()
        par = k % cpb
        xb = buf[k % _NBUF]
        pk = p_ref[:, par * _ROWS:(par + 1) * _ROWS]
        sk = jnp.dot(pk, xb, preferred_element_type=jnp.float32)
        nk = jnp.sum(xb * xb, axis=0)
        s = sk if par == 0 else s + sk
        norm2 = nk if par == 0 else norm2 + nk
        if k + _NBUF - 1 < nchunk:
            copy(k + _NBUF - 1).start()
        if par == cpb - 1:
            ev_f = jnp.max(s[:16], axis=0)
            ev_a = jnp.max(s[16:], axis=0)
            norm = jnp.maximum(jnp.sqrt(norm2), _EPS)
            o_ref[k // cpb] = (ev_f - ev_a) / norm


def kernel(x, forgery_protos, authentic_protos):
    b, c, h, w = x.shape
    hw = h * w
    protos = jnp.concatenate([forgery_protos, authentic_protos], axis=0)
    x2 = x.reshape(b * c, hw)

    out = pl.pallas_call(
        _guidance_kernel,
        in_specs=[
            pl.BlockSpec((protos.shape[0], c), lambda: (0, 0)),
            pl.BlockSpec(memory_space=pltpu.MemorySpace.HBM),
        ],
        out_specs=pl.BlockSpec((b, hw), lambda: (0, 0)),
        out_shape=jax.ShapeDtypeStruct((b, hw), jnp.float32),
        scratch_shapes=[
            pltpu.VMEM((_NBUF, _ROWS, hw), jnp.float32),
            pltpu.SemaphoreType.DMA((_NBUF,)),
        ],
    )(protos, x2)

    return out.reshape(b, 1, h, w)


# speedup vs baseline: 1.0142x; 1.0142x over previous
import jax
import jax.numpy as jnp
from jax.experimental import pallas as pl
from jax.experimental.pallas import tpu as pltpu

_EPS = 1e-12

_NBUF = 4
_ROWS = 256


def _guidance_kernel(p_ref, x_ref, o_ref, buf, sem):
    c = p_ref.shape[1]
    nrows_total = x_ref.shape[0]
    nchunk = nrows_total // _ROWS
    cpb = c // _ROWS

    def copy(k):
        return pltpu.make_async_copy(
            x_ref.at[pl.ds(k * _ROWS, _ROWS), :],
            buf.at[k % _NBUF],
            sem.at[k % _NBUF],
        )

    for k in range(min(_NBUF - 1, nchunk)):
        copy(k).start()

    s = None
    norm2 = None
    for k in range(nchunk):
        copy(k).wait()
        par = k % cpb
        xb = buf[k % _NBUF]
        pk = p_ref[:, par * _ROWS:(par + 1) * _ROWS]
        sk = jnp.dot(pk, xb, preferred_element_type=jnp.float32)
        nk = jnp.sum(xb * xb, axis=0)
        s = sk if par == 0 else s + sk
        norm2 = nk if par == 0 else norm2 + nk
        if k + _NBUF - 1 < nchunk:
            copy(k + _NBUF - 1).start()
        if par == cpb - 1:
            ev_f = jnp.max(s[:16], axis=0)
            ev_a = jnp.max(s[16:], axis=0)
            norm = jnp.maximum(jnp.sqrt(norm2), _EPS)
            o_ref[k // cpb] = (ev_f - ev_a) / norm


def kernel(x, forgery_protos, authentic_protos):
    b, c, h, w = x.shape
    hw = h * w
    protos = jnp.concatenate([forgery_protos, authentic_protos], axis=0)
    x2 = x.reshape(b * c, hw)

    out = pl.pallas_call(
        _guidance_kernel,
        in_specs=[
            pl.BlockSpec((protos.shape[0], c), lambda: (0, 0)),
            pl.BlockSpec(memory_space=pltpu.MemorySpace.HBM),
        ],
        out_specs=pl.BlockSpec((b, hw), lambda: (0, 0)),
        out_shape=jax.ShapeDtypeStruct((b, hw), jnp.float32),
        scratch_shapes=[
            pltpu.VMEM((_NBUF, _ROWS, hw), jnp.float32),
            pltpu.SemaphoreType.DMA((_NBUF,)),
        ],
    )(protos, x2)

    return out.reshape(b, 1, h, w)


# speedup vs baseline: 1.6593x; 1.6360x over previous
import jax
import jax.numpy as jnp
from jax import lax
from jax.experimental import pallas as pl
from jax.experimental.pallas import tpu as pltpu
from jax.experimental.pallas import tpu_sc as plsc

_EPS = 1e-12

_BBLK = 2
_NSC = 2
_NLANE = 16
_NPROTO = 16


def _tc_kernel(p_ref, x_ref, o_ref):
    for bi in range(_BBLK):
        xb = x_ref[bi]
        s = jnp.dot(p_ref[...], xb, preferred_element_type=jnp.float32)
        ev_f = jnp.max(s[:_NPROTO], axis=0)
        ev_a = jnp.max(s[_NPROTO:], axis=0)
        norm2 = jnp.sum(xb * xb, axis=0)
        norm = jnp.maximum(jnp.sqrt(norm2), _EPS)
        o_ref[bi] = ((ev_f - ev_a) / norm)[None, :]


def _rsqrt16(v):
    i = lax.bitcast_convert_type(v, jnp.int32)
    i = jnp.int32(0x5F3759DF) - lax.shift_right_logical(i, 1)
    y = lax.bitcast_convert_type(i, jnp.float32)
    for _ in range(4):
        y = y * (1.5 - 0.5 * v * y * y)
    return y


def _sc_kernel_body(x_hbm, pt_hbm, o_hbm, ptb, xb0, xb1, ob, sem0, sem1, psem):
    c_dim = pt_hbm.shape[0]
    pxw = ob.shape[0]
    ngrp = pxw // _NLANE
    wid = lax.axis_index("s") * 2 + lax.axis_index("c")
    base = wid * pxw

    pltpu.async_copy(pt_hbm, ptb, psem).wait()
    bufs = (xb0, xb1)
    sems = (sem0, sem1)
    b0 = x_hbm.shape[0] - _NSC
    cp0 = pltpu.async_copy(x_hbm.at[b0, :, pl.ds(base, pxw)], xb0, sem0)
    cp0.start()
    for img in range(_NSC):
        (cp0 if img == 0 else cp1).wait()
        if img + 1 < _NSC:
            cp1 = pltpu.async_copy(
                x_hbm.at[b0 + img + 1, :, pl.ds(base, pxw)],
                bufs[(img + 1) % 2], sems[(img + 1) % 2])
            cp1.start()
        xb = bufs[img % 2]
        for g in range(ngrp):
            sl = slice(g * _NLANE, (g + 1) * _NLANE)

            def pass_bank(bank, with_norm):
                def body(ci, carry):
                    xv = xb[ci, sl]
                    pv = ptb[ci, bank * _NPROTO:(bank + 1) * _NPROTO]
                    accs = tuple(
                        carry[p] + xv * pv[p]
                        for p in range(_NPROTO))
                    if with_norm:
                        return accs + (carry[_NPROTO] + xv * xv,)
                    return accs
                n_carry = _NPROTO + (1 if with_norm else 0)
                init = tuple(jnp.zeros((_NLANE,), jnp.float32)
                             for _ in range(n_carry))
                return lax.fori_loop(0, c_dim, body, init)

            res_f = pass_bank(0, True)
            res_a = pass_bank(1, False)
            ev_f = res_f[0]
            for p in range(1, _NPROTO):
                ev_f = jnp.maximum(ev_f, res_f[p])
            ev_a = res_a[0]
            for p in range(1, _NPROTO):
                ev_a = jnp.maximum(ev_a, res_a[p])
            n2 = jnp.maximum(res_f[_NPROTO], jnp.float32(_EPS * _EPS))
            ob[sl] = (ev_f - ev_a) * _rsqrt16(n2)
        pltpu.sync_copy(ob, o_hbm.at[img, pl.ds(base, pxw)])


def kernel(x, forgery_protos, authentic_protos):
    b, c, h, w = x.shape
    hw = h * w
    protos = jnp.concatenate([forgery_protos, authentic_protos], axis=0)
    x3 = x.reshape(b, c, hw)

    b_tc = b - _NSC

    tc_out = pl.pallas_call(
        _tc_kernel,
        grid=(b_tc // _BBLK,),
        in_specs=[
            pl.BlockSpec((protos.shape[0], c), lambda i: (0, 0)),
            pl.BlockSpec((_BBLK, c, hw), lambda i: (i, 0, 0)),
        ],
        out_specs=pl.BlockSpec((_BBLK, 1, hw), lambda i: (i, 0, 0)),
        out_shape=jax.ShapeDtypeStruct((b_tc, 1, hw), jnp.float32),
        compiler_params=pltpu.CompilerParams(
            dimension_semantics=("parallel",),
        ),
    )(protos, x3)

    pxw = hw // 32
    mesh = plsc.VectorSubcoreMesh(core_axis_name="c", subcore_axis_name="s")
    sc_fn = pl.kernel(
        _sc_kernel_body,
        out_type=jax.ShapeDtypeStruct((_NSC, hw), jnp.float32),
        mesh=mesh,
        scratch_types=[
            pltpu.VMEM((c, 2 * _NPROTO), jnp.float32),
            pltpu.VMEM((c, pxw), jnp.float32),
            pltpu.VMEM((c, pxw), jnp.float32),
            pltpu.VMEM((pxw,), jnp.float32),
            pltpu.SemaphoreType.DMA,
            pltpu.SemaphoreType.DMA,
            pltpu.SemaphoreType.DMA,
        ],
    )
    sc_out = sc_fn(x3, protos.T)

    out = jnp.concatenate([tc_out.reshape(b_tc, hw), sc_out], axis=0)
    return out.reshape(b, 1, h, w)


# speedup vs baseline: 1.6613x; 1.0012x over previous
import jax
import jax.numpy as jnp
from jax import lax
from jax.experimental import pallas as pl
from jax.experimental.pallas import tpu as pltpu
from jax.experimental.pallas import tpu_sc as plsc

_EPS = 1e-12

_BBLK = 2
_NSC = 2
_NLANE = 16
_NPROTO = 16


def _tc_kernel(p_ref, x_ref, o_ref):
    for bi in range(_BBLK):
        xb = x_ref[bi]
        s = jnp.dot(p_ref[...], xb, preferred_element_type=jnp.float32)
        ev_f = jnp.max(s[:_NPROTO], axis=0)
        ev_a = jnp.max(s[_NPROTO:], axis=0)
        norm2 = jnp.sum(xb * xb, axis=0)
        norm = jnp.maximum(jnp.sqrt(norm2), _EPS)
        o_ref[bi] = ((ev_f - ev_a) / norm)[None, :]


def _rsqrt16(v):
    i = lax.bitcast_convert_type(v, jnp.int32)
    i = jnp.int32(0x5F3759DF) - lax.shift_right_logical(i, 1)
    y = lax.bitcast_convert_type(i, jnp.float32)
    for _ in range(4):
        y = y * (1.5 - 0.5 * v * y * y)
    return y


def _sc_kernel_body(x_hbm, pt_hbm, o_hbm, ptb, xb0, xb1, ob, sem0, sem1, psem):
    c_dim = pt_hbm.shape[0]
    pxw = ob.shape[0]
    ngrp = pxw // _NLANE
    wid = lax.axis_index("s") * 2 + lax.axis_index("c")
    base = wid * pxw

    pltpu.async_copy(pt_hbm, ptb, psem).wait()
    bufs = (xb0, xb1)
    sems = (sem0, sem1)
    b0 = x_hbm.shape[0] - _NSC
    cp0 = pltpu.async_copy(x_hbm.at[b0, :, pl.ds(base, pxw)], xb0, sem0)
    cp0.start()
    for img in range(_NSC):
        (cp0 if img == 0 else cp1).wait()
        if img + 1 < _NSC:
            cp1 = pltpu.async_copy(
                x_hbm.at[b0 + img + 1, :, pl.ds(base, pxw)],
                bufs[(img + 1) % 2], sems[(img + 1) % 2])
            cp1.start()
        xb = bufs[img % 2]
        for g in range(ngrp):
            sl = slice(g * _NLANE, (g + 1) * _NLANE)

            def pass_bank(bank, with_norm):
                def body(ci, carry):
                    xv = xb[ci, sl]
                    pv = ptb[ci, bank * _NPROTO:(bank + 1) * _NPROTO]
                    accs = tuple(
                        carry[p] + xv * pv[p]
                        for p in range(_NPROTO))
                    if with_norm:
                        return accs + (carry[_NPROTO] + xv * xv,)
                    return accs
                n_carry = _NPROTO + (1 if with_norm else 0)
                init = tuple(jnp.zeros((_NLANE,), jnp.float32)
                             for _ in range(n_carry))
                return lax.fori_loop(0, c_dim, body, init)

            res_f = pass_bank(0, True)
            res_a = pass_bank(1, False)
            ev_f = res_f[0]
            for p in range(1, _NPROTO):
                ev_f = jnp.maximum(ev_f, res_f[p])
            ev_a = res_a[0]
            for p in range(1, _NPROTO):
                ev_a = jnp.maximum(ev_a, res_a[p])
            n2 = jnp.maximum(res_f[_NPROTO], jnp.float32(_EPS * _EPS))
            ob[sl] = (ev_f - ev_a) * _rsqrt16(n2)
        pltpu.sync_copy(ob, o_hbm.at[img, pl.ds(base, pxw)])


def kernel(x, forgery_protos, authentic_protos):
    b, c, h, w = x.shape
    hw = h * w
    protos = jnp.concatenate([forgery_protos, authentic_protos], axis=0)
    x3 = x.reshape(b, c, hw)

    b_tc = b - _NSC

    pxw = hw // 32
    mesh = plsc.VectorSubcoreMesh(core_axis_name="c", subcore_axis_name="s")
    sc_fn = pl.kernel(
        _sc_kernel_body,
        out_type=jax.ShapeDtypeStruct((_NSC, hw), jnp.float32),
        mesh=mesh,
        scratch_types=[
            pltpu.VMEM((c, 2 * _NPROTO), jnp.float32),
            pltpu.VMEM((c, pxw), jnp.float32),
            pltpu.VMEM((c, pxw), jnp.float32),
            pltpu.VMEM((pxw,), jnp.float32),
            pltpu.SemaphoreType.DMA,
            pltpu.SemaphoreType.DMA,
            pltpu.SemaphoreType.DMA,
        ],
    )
    sc_out = sc_fn(x3, protos.T)

    tc_out = pl.pallas_call(
        _tc_kernel,
        grid=(b_tc // _BBLK,),
        in_specs=[
            pl.BlockSpec((protos.shape[0], c), lambda i: (0, 0)),
            pl.BlockSpec((_BBLK, c, hw), lambda i: (i, 0, 0)),
        ],
        out_specs=pl.BlockSpec((_BBLK, 1, hw), lambda i: (i, 0, 0)),
        out_shape=jax.ShapeDtypeStruct((b_tc, 1, hw), jnp.float32),
        compiler_params=pltpu.CompilerParams(
            dimension_semantics=("parallel",),
        ),
    )(protos, x3)

    out = jnp.concatenate([tc_out.reshape(b_tc, hw), sc_out], axis=0)
    return out.reshape(b, 1, h, w)
